# baseline (speedup 1.0000x reference)
import functools

import jax
import jax.numpy as jnp
from jax import lax
from jax.experimental import pallas as pl
from jax.experimental.pallas import tpu as pltpu
from jax.experimental.pallas import tpu_sc as plsc

BATCH = 16384
EMBED = 32
HIDDEN = 64
NUM_U = 1000000
NUM_M = 100000

_NC = 2
_NS = 16
_NW = _NC * _NS
_LANES = 128

_W = 1024
_NGRP = BATCH // 16
_DUMMY = BATCH
_OUT_ROWS = BATCH + 256
_STG = 64


def _ranges(nrows):
  tiles = -(-nrows // _LANES)
  tpw = -(-tiles // _NW)
  range_cols = tpw * _LANES
  nch = -(-range_cols // _W)
  clamp = (tiles - _W // _LANES) * _LANES
  return range_cols, nch, clamp


_RANGE_U, _NCH_U, _CLAMP_U = _ranges(NUM_U)
_RANGE_M, _NCH_M, _CLAMP_M = _ranges(NUM_M)


def _sc_scan_gather(ut_t, mt_t, uidx, midx):
  mesh = plsc.VectorSubcoreMesh(core_axis_name="c", subcore_axis_name="s")

  @functools.partial(
      pl.kernel,
      mesh=mesh,
      out_type=[
          jax.ShapeDtypeStruct((_OUT_ROWS, _LANES), jnp.float32),
          jax.ShapeDtypeStruct((_OUT_ROWS, _LANES), jnp.float32),
      ],
      scratch_types=[
          pltpu.VMEM((BATCH,), jnp.int32),
          pltpu.VMEM((BATCH,), jnp.int32),
          pltpu.VMEM((BATCH,), jnp.int32),
          pltpu.VMEM((2, EMBED, _W), jnp.float32),
          pltpu.VMEM((_STG, _LANES), jnp.float32),
          pltpu.VMEM((1, _STG), jnp.int32),
          pltpu.SemaphoreType.DMA,
          pltpu.SemaphoreType.DMA,
          pltpu.SemaphoreType.DMA,
      ],
      compiler_params=pltpu.CompilerParams(
          use_tc_tiling_on_sc=True, needs_layout_passes=False),
  )
  def k(ut_hbm, mt_hbm, uidx_hbm, midx_hbm, out_u, out_m,
        idx_v, hup, hmp, bufs, stg_f, stg_p, sem0, sem1, semf):
    wid = lax.axis_index("s") * _NC + lax.axis_index("c")
    iota = lax.iota(jnp.int32, 16)
    zeros16 = jnp.zeros((16,), jnp.float32)
    dummy16 = jnp.full((16,), _DUMMY, jnp.int32)

    for r in range(_STG):
      for t in range(EMBED // 16, _LANES // 16):
        stg_f[r, pl.ds(t * 16, 16)] = zeros16
    for t in range(_STG // 16):
      stg_p[0, pl.ds(t * 16, 16)] = dummy16

    def phase_a(idx_hbm, hp, lo, range_cols):
      pltpu.sync_copy(idx_hbm, idx_v)
      hi = lo + range_cols

      def body(g, cnt):
        off = pl.multiple_of(g * 16, 8)
        v = idx_v[pl.ds(off, 16)]
        m = (v >= lo) & (v < hi)
        mi = jnp.where(m, 1, 0)
        tot = lax.reduce_sum(mi, axes=(0,))

        @pl.when(tot > 0)
        def _():
          dest = cnt + plsc.cumsum(mi) - 1
          packed = ((v - lo) << 14) | (g * 16 + iota)
          plsc.store_scatter(hp, [dest], packed, mask=m)

        return cnt + tot

      return lax.fori_loop(0, _NGRP, body, jnp.int32(0))

    def reset_stg_p():
      for t in range(_STG // 16):
        stg_p[0, pl.ds(t * 16, 16)] = dummy16

    def phase_b(tbl_hbm, hp, cnt, lo, nch, clamp, out, sems, pend):
      ngrp = (cnt + 15) // 16
      mask0 = iota == 0
      zeros_i = jnp.zeros((16,), jnp.int32)

      def c0_of(kk):
        c0 = jnp.minimum(lo + kk * _W, clamp)
        return pl.multiple_of(c0, _LANES)

      def compact(c0_rel):
        def body(j, cnt2):
          goff = pl.multiple_of(j * 16, 8)
          x = hp[pl.ds(goff, 16)]
          hrel = lax.shift_right_logical(x, 14)
          valid = (goff + iota) < cnt
          m = (hrel >= c0_rel) & (hrel < c0_rel + _W) & valid
          mi = jnp.where(m, 1, 0)
          tot = lax.reduce_sum(mi, axes=(0,))

          @pl.when(tot > 0)
          def _():
            dest = cnt2 + plsc.cumsum(mi) - 1
            plsc.store_scatter(idx_v, [dest], x, mask=m)

          return cnt2 + tot

        return lax.fori_loop(0, ngrp, body, jnp.int32(0))

      def extract(buf_ref, c0_rel, cnt2, s_idx):
        def body(j2, s_idx):
          @pl.when(s_idx == _STG)
          def _():
            pltpu.async_copy(stg_f, out.at[stg_p.at[0]], semf).wait()
            reset_stg_p()

          s_idx = jnp.where(s_idx == _STG, 0, s_idx)
          gbase = j2 * 16

          def hit(i, _):
            xi = plsc.load_gather(
                idx_v, [jnp.full((16,), gbase + i, jnp.int32)])
            col = jnp.clip(
                lax.shift_right_logical(xi, 14) - c0_rel, 0, _W - 1)
            pv = jnp.where(gbase + i < cnt2, xi & (BATCH - 1), _DUMMY)
            g1 = plsc.load_gather(buf_ref, [iota, col])
            g2 = plsc.load_gather(buf_ref, [iota + 16, col])
            stg_f[s_idx + i, pl.ds(0, 16)] = g1
            stg_f[s_idx + i, pl.ds(16, 16)] = g2
            plsc.store_scatter(
                stg_p, [zeros_i, jnp.full((16,), s_idx + i, jnp.int32)],
                pv, mask=mask0)
            return 0

          lax.fori_loop(0, 16, hit, 0)
          return s_idx + 16

        ngrp2 = (cnt2 + 15) // 16
        return lax.fori_loop(0, ngrp2, body, s_idx)

      s_idx = jnp.int32(0)
      for kk in range(nch):
        b = kk % 2
        nxt = None
        if kk + 1 < nch:
          nxt = pltpu.async_copy(
              tbl_hbm.at[:, pl.ds(c0_of(kk + 1), _W)],
              bufs.at[(kk + 1) % 2], sems[(kk + 1) % 2])
        pend.wait()
        c0_rel = c0_of(kk) - lo
        cnt2 = compact(c0_rel)
        s_idx = extract(bufs.at[b], c0_rel, cnt2, s_idx)
        pend = nxt

      @pl.when(s_idx > 0)
      def _():
        pltpu.async_copy(stg_f, out.at[stg_p.at[0]], semf).wait()
        reset_stg_p()

    def first_chunk(tbl_hbm, lo, clamp):
      c0 = pl.multiple_of(jnp.minimum(lo, clamp), _LANES)
      return pltpu.async_copy(
          tbl_hbm.at[:, pl.ds(c0, _W)], bufs.at[0], sem0)

    pend = first_chunk(ut_hbm, wid * _RANGE_U, _CLAMP_U)
    cnt_u = phase_a(uidx_hbm, hup, wid * _RANGE_U, _RANGE_U)
    phase_b(ut_hbm, hup, cnt_u, wid * _RANGE_U, _NCH_U, _CLAMP_U, out_u,
            (sem0, sem1), pend)
    pend = first_chunk(mt_hbm, wid * _RANGE_M, _CLAMP_M)
    cnt_m = phase_a(midx_hbm, hmp, wid * _RANGE_M, _RANGE_M)
    phase_b(mt_hbm, hmp, cnt_m, wid * _RANGE_M, _NCH_M, _CLAMP_M, out_m,
            (sem0, sem1), pend)

  return k(ut_t, mt_t, uidx, midx)


_BLK = 2048


def _mlp_body(u_ref, m_ref, w1u_ref, w1m_ref, b1_ref, w2_ref, b2_ref, o_ref):
  h = (jnp.dot(u_ref[...], w1u_ref[...], preferred_element_type=jnp.float32)
       + jnp.dot(m_ref[...], w1m_ref[...], preferred_element_type=jnp.float32)
       + b1_ref[...])
  h = jnp.maximum(h, 0.0)
  o_ref[...] = jnp.sum(h * w2_ref[...], axis=1) + b2_ref[...]


def _tc_mlp(u_g, m_g, W1u, W1m, b1, W2, b2):
  grid = (BATCH // _BLK,)
  return pl.pallas_call(
      _mlp_body,
      grid=grid,
      in_specs=[
          pl.BlockSpec((_BLK, _LANES), lambda i: (i, 0)),
          pl.BlockSpec((_BLK, _LANES), lambda i: (i, 0)),
          pl.BlockSpec((_LANES, HIDDEN), lambda i: (0, 0)),
          pl.BlockSpec((_LANES, HIDDEN), lambda i: (0, 0)),
          pl.BlockSpec((1, HIDDEN), lambda i: (0, 0)),
          pl.BlockSpec((1, HIDDEN), lambda i: (0, 0)),
          pl.BlockSpec((1,), lambda i: (0,)),
      ],
      out_specs=pl.BlockSpec((_BLK,), lambda i: (i,)),
      out_shape=jax.ShapeDtypeStruct((BATCH,), jnp.float32),
  )(u_g, m_g, W1u, W1m, b1.reshape(1, HIDDEN), W2.reshape(1, HIDDEN), b2)


def kernel(user_idx, movie_idx, user_table, movie_table, W1, b1, W2, b2):
  uidx = user_idx.astype(jnp.int32)
  midx = movie_idx.astype(jnp.int32)
  u_g, m_g = _sc_scan_gather(user_table.T, movie_table.T, uidx, midx)
  W1u = jnp.zeros((_LANES, HIDDEN), jnp.float32).at[:EMBED].set(W1[:EMBED])
  W1m = jnp.zeros((_LANES, HIDDEN), jnp.float32).at[:EMBED].set(W1[EMBED:])
  return _tc_mlp(u_g, m_g, W1u, W1m, b1, W2, b2)


# speedup vs baseline: 2.5225x; 2.5225x over previous
import functools

import jax
import jax.numpy as jnp
from jax import lax
from jax.experimental import pallas as pl
from jax.experimental.pallas import tpu as pltpu
from jax.experimental.pallas import tpu_sc as plsc

BATCH = 16384
EMBED = 32
HIDDEN = 64
NUM_U = 1000000
NUM_M = 100000

_NC = 2
_NS = 16
_NW = _NC * _NS
_LANES = 128

_W = 1024
_NGRP = BATCH // 16
_DUMMY = BATCH
_OUT_ROWS = BATCH + 256
_STG = 64


def _ranges(nrows):
  tiles = -(-nrows // _LANES)
  tpw = -(-tiles // _NW)
  range_cols = tpw * _LANES
  nch = -(-range_cols // _W)
  clamp = (tiles - _W // _LANES) * _LANES
  return range_cols, nch, clamp


_RANGE_U, _NCH_U, _CLAMP_U = _ranges(NUM_U)
_RANGE_M, _NCH_M, _CLAMP_M = _ranges(NUM_M)


def _sc_scan_gather(ut_t, mt_t, uidx, midx):
  mesh = plsc.VectorSubcoreMesh(core_axis_name="c", subcore_axis_name="s")

  @functools.partial(
      pl.kernel,
      mesh=mesh,
      out_type=[
          jax.ShapeDtypeStruct((_OUT_ROWS, _LANES), jnp.float32),
          jax.ShapeDtypeStruct((_OUT_ROWS, _LANES), jnp.float32),
      ],
      scratch_types=[
          pltpu.VMEM((BATCH,), jnp.int32),
          pltpu.VMEM((BATCH,), jnp.int32),
          pltpu.VMEM((BATCH,), jnp.int32),
          pltpu.VMEM((2, EMBED, _W), jnp.float32),
          pltpu.VMEM((_STG, _LANES), jnp.float32),
          pltpu.VMEM((1, _STG), jnp.int32),
          pltpu.SemaphoreType.DMA,
          pltpu.SemaphoreType.DMA,
          pltpu.SemaphoreType.DMA,
      ],
      compiler_params=pltpu.CompilerParams(
          use_tc_tiling_on_sc=True, needs_layout_passes=False),
  )
  def k(ut_hbm, mt_hbm, uidx_hbm, midx_hbm, out_u, out_m,
        idx_v, hup, hmp, bufs, stg_f, stg_p, sem0, sem1, semf):
    wid = lax.axis_index("s") * _NC + lax.axis_index("c")
    iota = lax.iota(jnp.int32, 16)
    zeros16 = jnp.zeros((16,), jnp.float32)
    dummy16 = jnp.full((16,), _DUMMY, jnp.int32)

    for r in range(_STG):
      for t in range(EMBED // 16, _LANES // 16):
        stg_f[r, pl.ds(t * 16, 16)] = zeros16
    for t in range(_STG // 16):
      stg_p[0, pl.ds(t * 16, 16)] = dummy16

    def phase_a(idx_hbm, hp, lo, range_cols):
      pltpu.sync_copy(idx_hbm, idx_v)
      hi = lo + range_cols

      def body(g, cnt):
        off = pl.multiple_of(g * 16, 8)
        v = idx_v[pl.ds(off, 16)]
        m = (v >= lo) & (v < hi)
        mi = jnp.where(m, 1, 0)
        tot = lax.reduce_sum(mi, axes=(0,))

        @pl.when(tot > 0)
        def _():
          dest = cnt + plsc.cumsum(mi) - 1
          packed = ((v - lo) << 14) | (g * 16 + iota)
          plsc.store_scatter(hp, [dest], packed, mask=m)

        return cnt + tot

      return lax.fori_loop(0, _NGRP, body, jnp.int32(0))

    def reset_stg_p():
      for t in range(_STG // 16):
        stg_p[0, pl.ds(t * 16, 16)] = dummy16

    def phase_b(tbl_hbm, hp, cnt, lo, nch, clamp, out, sems, pend):
      ngrp = (cnt + 15) // 16
      mask0 = iota == 0
      zeros_i = jnp.zeros((16,), jnp.int32)

      def c0_of(kk):
        c0 = jnp.minimum(lo + kk * _W, clamp)
        return pl.multiple_of(c0, _LANES)

      def group(buf_ref, c0_rel):
        def body(j, s_idx):
          goff = pl.multiple_of(j * 16, 8)
          x = hp[pl.ds(goff, 16)]
          h = lax.shift_right_logical(x, 14)
          valid = (goff + iota) < cnt
          m0 = jnp.where((h >= c0_rel) & (h < c0_rel + _W) & valid, 1, 0)

          def w_cond(carry):
            m, _ = carry
            return lax.reduce_sum(m, axes=(0,)) > 0

          def w_body(carry):
            m, s_idx = carry
            mb = m > 0
            lane = plsc.all_reduce_ffs(mb)
            onehot = iota == lane
            xv = lax.reduce_sum(jnp.where(onehot, x, 0), axes=(0,))
            xs = jnp.full((16,), xv, jnp.int32)
            colv = lax.shift_right_logical(xs, 14) - c0_rel
            g1 = plsc.load_gather(buf_ref, [iota, colv])
            g2 = plsc.load_gather(buf_ref, [iota + 16, colv])
            stg_f[s_idx, pl.ds(0, 16)] = g1
            stg_f[s_idx, pl.ds(16, 16)] = g2
            plsc.store_scatter(
                stg_p, [zeros_i, jnp.full((16,), s_idx, jnp.int32)],
                xs & (BATCH - 1), mask=mask0)
            s_idx = s_idx + 1

            @pl.when(s_idx == _STG)
            def _():
              pltpu.async_copy(stg_f, out.at[stg_p.at[0]], semf).wait()
              reset_stg_p()

            s_idx = jnp.where(s_idx == _STG, 0, s_idx)
            return jnp.where(onehot, 0, m), s_idx

          _, s_idx = lax.while_loop(w_cond, w_body, (m0, s_idx))
          return s_idx

        return body

      s_idx = jnp.int32(0)
      for kk in range(nch):
        b = kk % 2
        nxt = None
        if kk + 1 < nch:
          nxt = pltpu.async_copy(
              tbl_hbm.at[:, pl.ds(c0_of(kk + 1), _W)],
              bufs.at[(kk + 1) % 2], sems[(kk + 1) % 2])
        pend.wait()
        s_idx = lax.fori_loop(
            0, ngrp, group(bufs.at[b], c0_of(kk) - lo), s_idx)
        pend = nxt

      @pl.when(s_idx > 0)
      def _():
        pltpu.async_copy(stg_f, out.at[stg_p.at[0]], semf).wait()
        reset_stg_p()

    def first_chunk(tbl_hbm, lo, clamp):
      c0 = pl.multiple_of(jnp.minimum(lo, clamp), _LANES)
      return pltpu.async_copy(
          tbl_hbm.at[:, pl.ds(c0, _W)], bufs.at[0], sem0)

    pend = first_chunk(ut_hbm, wid * _RANGE_U, _CLAMP_U)
    cnt_u = phase_a(uidx_hbm, hup, wid * _RANGE_U, _RANGE_U)
    phase_b(ut_hbm, hup, cnt_u, wid * _RANGE_U, _NCH_U, _CLAMP_U, out_u,
            (sem0, sem1), pend)
    pend = first_chunk(mt_hbm, wid * _RANGE_M, _CLAMP_M)
    cnt_m = phase_a(midx_hbm, hmp, wid * _RANGE_M, _RANGE_M)
    phase_b(mt_hbm, hmp, cnt_m, wid * _RANGE_M, _NCH_M, _CLAMP_M, out_m,
            (sem0, sem1), pend)

  return k(ut_t, mt_t, uidx, midx)


_BLK = 2048


def _mlp_body(u_ref, m_ref, w1u_ref, w1m_ref, b1_ref, w2_ref, b2_ref, o_ref):
  h = (jnp.dot(u_ref[...], w1u_ref[...], preferred_element_type=jnp.float32)
       + jnp.dot(m_ref[...], w1m_ref[...], preferred_element_type=jnp.float32)
       + b1_ref[...])
  h = jnp.maximum(h, 0.0)
  o_ref[...] = jnp.sum(h * w2_ref[...], axis=1) + b2_ref[...]


def _tc_mlp(u_g, m_g, W1u, W1m, b1, W2, b2):
  grid = (BATCH // _BLK,)
  return pl.pallas_call(
      _mlp_body,
      grid=grid,
      in_specs=[
          pl.BlockSpec((_BLK, _LANES), lambda i: (i, 0)),
          pl.BlockSpec((_BLK, _LANES), lambda i: (i, 0)),
          pl.BlockSpec((_LANES, HIDDEN), lambda i: (0, 0)),
          pl.BlockSpec((_LANES, HIDDEN), lambda i: (0, 0)),
          pl.BlockSpec((1, HIDDEN), lambda i: (0, 0)),
          pl.BlockSpec((1, HIDDEN), lambda i: (0, 0)),
          pl.BlockSpec((1,), lambda i: (0,)),
      ],
      out_specs=pl.BlockSpec((_BLK,), lambda i: (i,)),
      out_shape=jax.ShapeDtypeStruct((BATCH,), jnp.float32),
  )(u_g, m_g, W1u, W1m, b1.reshape(1, HIDDEN), W2.reshape(1, HIDDEN), b2)


def kernel(user_idx, movie_idx, user_table, movie_table, W1, b1, W2, b2):
  uidx = user_idx.astype(jnp.int32)
  midx = movie_idx.astype(jnp.int32)
  u_g, m_g = _sc_scan_gather(user_table.T, movie_table.T, uidx, midx)
  W1u = jnp.zeros((_LANES, HIDDEN), jnp.float32).at[:EMBED].set(W1[:EMBED])
  W1m = jnp.zeros((_LANES, HIDDEN), jnp.float32).at[:EMBED].set(W1[EMBED:])
  return _tc_mlp(u_g, m_g, W1u, W1m, b1, W2, b2)
